# baseline (speedup 1.0000x reference)
import functools

import jax
import jax.numpy as jnp
from jax import lax
from jax.experimental import pallas as pl
from jax.experimental.pallas import tpu as pltpu
from jax.experimental.pallas import tpu_sc as plsc

A0 = 0.529177249
SIG2 = [0.5515909**2, 1.8886297**2, 1.3225029**2, 1.2316629**2,
        2.1884933**2, 1.7750372**2, 1.3677907**2, 1.3820058**2]
NM, NA, NS = 128, 64, 8
G = 8
T = G * NA
STEPS = NM // G
NT = NM * NA
DF = 512
TB = 256
HP = jax.lax.Precision.HIGHEST


def _dot2(a, b):
    bh = b.astype(jnp.bfloat16).astype(jnp.float32)
    return jnp.dot(a, bh) + jnp.dot(a, b - bh)


def _dot2a(a, b):
    ah = a.astype(jnp.bfloat16).astype(jnp.float32)
    return jnp.dot(ah, b) + jnp.dot(a - ah, b)


def _celu(x):
    return jnp.maximum(x, 0.0) + 0.1 * (jnp.exp(jnp.minimum(x * 10.0, 0.0)) - 1.0)


def _softplus(x):
    return jnp.maximum(x, 0.0) + jnp.log(1.0 + jnp.exp(-jnp.abs(x)))


def _erf(x):
    t = 1.0 / (1.0 + 0.3275911 * x)
    p = t * (0.254829592 + t * (-0.284496736 + t * (1.421413741
              + t * (-1.453152027 + t * 1.061405429))))
    return 1.0 - p * jnp.exp(-x * x)


def _body_a(spc_ref, spr_ref, cc_ref, cr_ref, nq_ref, s2c_ref,
            w16a_ref, w16q_ref,
            c0a_ref, c0q_ref, c0qr_ref, c0er_ref, c1_ref, c2_ref, c3_ref,
            cb0_ref, cb1_ref, cb2_ref, cb3_ref,
            ft_ref, coul_ref, q_ref, rk_ref, cnt_ref,
            nbr_ref, jbd_ref, carry_ref):
    pid = pl.program_id(0)

    @pl.when(pid == 0)
    def _init():
        nbr_ref[...] = jnp.zeros((T, T), jnp.float32)
        jbd_ref[...] = jnp.zeros((T, T), jnp.float32)
        carry_ref[...] = jnp.zeros((1, NS), jnp.float32)

    sp_c = spc_ref[0]
    sp_r = spr_ref[0]
    onehot = (sp_c == jax.lax.broadcasted_iota(jnp.int32, (T, NS), 1)
              ).astype(jnp.float32)
    sig2_c = _dot2(onehot, s2c_ref[...])
    sig2_r = jnp.full((1, T), SIG2[0], jnp.float32)
    for e in range(1, NS):
        sig2_r = jnp.where(sp_r == e, jnp.float32(SIG2[e]), sig2_r)

    ii = jax.lax.broadcasted_iota(jnp.int32, (NA, NA), 0)
    jj = jax.lax.broadcasted_iota(jnp.int32, (NA, NA), 1)
    offm = jnp.where(ii == jj, 0.0, 1.0).astype(jnp.float32)

    for g in range(G):
        sl = pl.ds(g * NA, NA)
        d2 = jnp.full((NA, NA), 1e-16, jnp.float32)
        for ax in range(3):
            col = cc_ref[0, sl, ax:ax + 1]
            row = cr_ref[0, ax:ax + 1, sl]
            dif = col - row
            d2 = d2 + dif * dif
        dist = jnp.sqrt(d2) * jnp.float32(1.0 / A0)
        nbr_ref[sl, sl] = jnp.exp(-dist) * offm
        s2 = sig2_c[g * NA:(g + 1) * NA, :] + sig2_r[:, g * NA:(g + 1) * NA]
        x = dist * jax.lax.rsqrt(2.0 * s2)
        jbd_ref[sl, sl] = _erf(x) / dist * offm

    ti = jax.lax.broadcasted_iota(jnp.int32, (T, T), 0)
    tj = jax.lax.broadcasted_iota(jnp.int32, (T, T), 1)
    stri = jnp.where(tj < ti, 1.0, 0.0).astype(jnp.float32)
    carry = carry_ref[...]
    rank_tot = jnp.dot(stri, onehot) + carry
    rk_ref[0] = jnp.sum(rank_tot * onehot, axis=1,
                        keepdims=True).astype(jnp.int32)
    carry_new = carry + jnp.sum(onehot, axis=0, keepdims=True)
    carry_ref[...] = carry_new
    oe = jax.lax.broadcasted_iota(jnp.int32, (NS, 16), 0)
    oj = jax.lax.broadcasted_iota(jnp.int32, (NS, 16), 1)
    mlt = jnp.where(oe < oj, 1.0, 0.0).astype(jnp.float32)
    cnt_ref[...] = _dot2a(carry_new, mlt).astype(jnp.int32)

    gi = jax.lax.broadcasted_iota(jnp.int32, (G, T), 0)
    gt = jax.lax.broadcasted_iota(jnp.int32, (G, T), 1)
    seg = jnp.where(gt // NA == gi, 1.0, 0.0).astype(jnp.float32)
    si = jax.lax.broadcasted_iota(jnp.int32, (T, G), 0)
    sj = jax.lax.broadcasted_iota(jnp.int32, (T, G), 1)
    segT = jnp.where(si // NA == sj, 1.0, 0.0).astype(jnp.float32)

    base16 = jnp.concatenate([cc_ref[0], onehot], axis=1)
    phi_aev = jnp.tanh(jnp.dot(base16, w16a_ref[...]))
    aev = jnp.tanh(jnp.dot(nbr_ref[...], phi_aev))

    c1 = c1_ref[...]; c2 = c2_ref[...]; c3 = c3_ref[...]
    cb1 = cb1_ref[...]; cb2 = cb2_ref[...]; cb3 = cb3_ref[...]
    nq = nq_ref[0]

    def chi_tail(pre):
        h = _celu(pre)
        h = _celu(jnp.dot(h, c1) + cb1)
        h = _celu(jnp.dot(h, c2) + cb2)
        return _softplus(jnp.dot(h, c3) + cb3)

    def equil(chi):
        sums = _dot2(seg, chi)
        k_net = 1.0 + jnp.abs(nq) / sums
        k_p = jnp.where(nq > 0, k_net, 1.0)
        k_n = jnp.where(nq < 0, k_net, 1.0)
        ktab = jnp.concatenate(
            [k_n, k_p * (sums * jnp.float32(1.0 / NA))], axis=1)
        kexp = _dot2(segT, ktab)
        return -kexp[:, 0:1] * chi + kexp[:, 1:2]

    def esp_of(q):
        return _dot2(jbd_ref[...], q)

    h_aev = jnp.dot(aev, c0a_ref[...]) + cb0_ref[...]

    chi1 = chi_tail(h_aev)
    q1 = equil(chi1)
    esp1 = esp_of(q1)

    phi_qr = jnp.tanh(jnp.dot(base16, w16q_ref[...]))
    qraev = jnp.tanh(jnp.dot(nbr_ref[...], q1 * phi_qr))

    pre2 = (h_aev + jnp.dot(qraev, c0q_ref[...])
            + q1 * c0qr_ref[...] + esp1 * c0er_ref[...])
    chi2 = chi_tail(pre2)
    q2 = equil(chi2)
    esp2 = esp_of(q2)

    molid = (G * pid + jax.lax.broadcasted_iota(jnp.int32, (T, 1), 0) // NA
             ).astype(jnp.float32)
    ft_ref[0] = jnp.concatenate(
        [aev, qraev, q2, esp2, molid,
         jnp.zeros((T, DF - 451), jnp.float32)], axis=1)

    coul_ref[0] = 0.5 * _dot2(seg, q2 * esp2)
    q_ref[0] = q2


def _run_a(species, coordinates, net_charge, params):
    sp_col = species.reshape(STEPS, T, 1)
    sp_row = species.reshape(STEPS, 1, T)
    cf = coordinates.reshape(STEPS, T, 3)
    coords_c = jnp.pad(cf, ((0, 0), (0, 0), (0, 5)))
    coords_r = jnp.pad(cf.transpose(0, 2, 1), ((0, 0), (0, 5), (0, 0)))
    netq = net_charge.reshape(STEPS, G, 1)
    sig2 = jnp.asarray(SIG2, jnp.float32).reshape(NS, 1)

    p = params
    w16a = jnp.concatenate(
        [jnp.pad(p['W_aev'][:3], ((0, 5), (0, 0))), p['W_aev'][3:]], axis=0)
    w16q = jnp.concatenate(
        [jnp.pad(p['W_qr'][:3], ((0, 5), (0, 0))), p['W_qr'][3:]], axis=0)
    c0 = p['chi_W0']
    c0a, c0q = c0[:384], c0[384:448]
    c0qr, c0er = c0[448:449], c0[449:450]

    def bs(a):
        nd = a.ndim
        return pl.BlockSpec(a.shape, lambda i, _n=nd: (0,) * _n)

    ins = [sp_col, sp_row, coords_c, coords_r, netq, sig2,
           w16a, w16q,
           c0a, c0q, c0qr, c0er, p['chi_W1'], p['chi_W2'], p['chi_W3'],
           p['chi_b0'].reshape(1, -1), p['chi_b1'].reshape(1, -1),
           p['chi_b2'].reshape(1, -1), p['chi_b3'].reshape(1, -1)]

    specs = [pl.BlockSpec((1, T, 1), lambda i: (i, 0, 0)),
             pl.BlockSpec((1, 1, T), lambda i: (i, 0, 0)),
             pl.BlockSpec((1, T, 8), lambda i: (i, 0, 0)),
             pl.BlockSpec((1, 8, T), lambda i: (i, 0, 0)),
             pl.BlockSpec((1, G, 1), lambda i: (i, 0, 0))]
    specs += [bs(a) for a in ins[5:]]

    out_shapes = (jax.ShapeDtypeStruct((STEPS, T, DF), jnp.float32),
                  jax.ShapeDtypeStruct((STEPS, G, 1), jnp.float32),
                  jax.ShapeDtypeStruct((STEPS, T, 1), jnp.float32),
                  jax.ShapeDtypeStruct((STEPS, T, 1), jnp.int32),
                  jax.ShapeDtypeStruct((1, 16), jnp.int32))
    out_specs = (pl.BlockSpec((1, T, DF), lambda i: (i, 0, 0)),
                 pl.BlockSpec((1, G, 1), lambda i: (i, 0, 0)),
                 pl.BlockSpec((1, T, 1), lambda i: (i, 0, 0)),
                 pl.BlockSpec((1, T, 1), lambda i: (i, 0, 0)),
                 pl.BlockSpec((1, 16), lambda i: (0, 0)))

    return pl.pallas_call(
        _body_a,
        grid=(STEPS,),
        in_specs=specs,
        out_specs=out_specs,
        out_shape=out_shapes,
        scratch_shapes=[pltpu.VMEM((T, T), jnp.float32),
                        pltpu.VMEM((T, T), jnp.float32),
                        pltpu.VMEM((1, NS), jnp.float32)],
    )(*ins)


def _sc_dispatch(feats, species_flat, rank_i, offs16):
    mesh = plsc.VectorSubcoreMesh(core_axis_name="c", subcore_axis_name="s")
    NB, RB = 4, 64

    @functools.partial(
        pl.kernel, mesh=mesh,
        out_type=jax.ShapeDtypeStruct((NT, DF), jnp.float32),
        compiler_params=pltpu.CompilerParams(needs_layout_passes=False),
        scratch_types=[
            pltpu.VMEM((TB,), jnp.int32),
            pltpu.VMEM((TB,), jnp.int32),
            pltpu.VMEM((16,), jnp.int32),
            pltpu.VMEM((NB, RB), jnp.int32),
            pltpu.VMEM((3, RB, DF), jnp.float32),
            pltpu.SemaphoreType.DMA,
            pltpu.SemaphoreType.DMA,
        ])
    def k(ft_hbm, sp_hbm, rk_hbm, off_hbm, out_hbm,
          sp_v, rk_v, off_v, dest_v, rows_v, sem_i, sem_o):
        wid = lax.axis_index("s") * 2 + lax.axis_index("c")
        base = wid * TB
        ih, oh = {}, {}

        def start_in(b):
            ih[b] = pltpu.async_copy(
                ft_hbm.at[pl.ds(base + b * RB, RB)], rows_v.at[b % 3], sem_i)

        for b in range(3):
            start_in(b)
        pltpu.sync_copy(sp_hbm.at[pl.ds(base, TB)], sp_v)
        pltpu.sync_copy(rk_hbm.at[pl.ds(base, TB)], rk_v)
        pltpu.sync_copy(off_hbm, off_v)
        for j in range(16):
            s = sp_v[pl.ds(j * 16, 16)]
            r = rk_v[pl.ds(j * 16, 16)]
            o = plsc.load_gather(off_v, [s])
            dest_v[j // 4, pl.ds((j % 4) * 16, 16)] = o + r
        for b in range(NB):
            ih[b].wait()
            oh[b] = pltpu.async_copy(
                rows_v.at[b % 3], out_hbm.at[dest_v.at[b]], sem_o)
            if b + 3 < NB:
                oh[b].wait()
                start_in(b + 3)
        for b in range(NB):
            if not (b + 3 < NB):
                oh[b].wait()

    return k(feats, species_flat, rank_i, offs16)


def _body_b(sf_ref, cnt_ref, coul_ref,
            w0_ref, w1_ref, w2_ref, w3_ref,
            b0_ref, b1_ref, b2_ref, b3_ref,
            out_ref):
    b = pl.program_id(0)

    @pl.when(b == 0)
    def _init():
        out_ref[...] = coul_ref[...]

    sf = sf_ref[...]
    molid = sf[:, 450:451]

    offs = cnt_ref[...].astype(jnp.float32)[:, 0:NS + 1]
    glob = (TB * b + jax.lax.broadcasted_iota(jnp.int32, (TB, 1), 0)
            ).astype(jnp.float32)
    lo_f = jnp.float32(TB) * b.astype(jnp.float32)
    hi_f = lo_f + jnp.float32(TB - 1)
    off8 = offs[:, 0:NS]
    e_lo = (jnp.sum(jnp.where(off8 <= lo_f, 1.0, 0.0)) - 1.0).astype(jnp.int32)
    e_hi = (jnp.sum(jnp.where(off8 <= hi_f, 1.0, 0.0)) - 1.0).astype(jnp.int32)

    def expert(i, acc):
        e = e_lo + i
        seg_lo = jnp.sum(jnp.where(
            jax.lax.broadcasted_iota(jnp.int32, (1, NS + 1), 1) == e,
            offs, 0.0))
        seg_hi = jnp.sum(jnp.where(
            jax.lax.broadcasted_iota(jnp.int32, (1, NS + 1), 1) == e + 1,
            offs, 0.0))
        msk = jnp.logical_and(glob >= seg_lo, glob < seg_hi)
        h = _celu(jnp.dot(sf, w0_ref[e]) + b0_ref[e])
        h = _celu(jnp.dot(h, w1_ref[e]) + b1_ref[e])
        h = _celu(jnp.dot(h, w2_ref[e]) + b2_ref[e])
        o = jnp.dot(h, w3_ref[e]) + b3_ref[e]
        return acc + jnp.where(msk, o, 0.0)

    en = jax.lax.fori_loop(0, e_hi - e_lo + 1, expert,
                           jnp.zeros((TB, 1), jnp.float32))
    mi = jax.lax.broadcasted_iota(jnp.int32, (TB, NM), 1).astype(jnp.float32)
    oh = jnp.where(molid == mi, 1.0, 0.0).astype(jnp.float32)
    out_ref[...] += jnp.sum(en * oh, axis=0, keepdims=True)


def _run_b(sorted_feats, offs, coul, params):
    p = params
    w0 = jnp.concatenate(
        [p['ani_W0'], jnp.zeros((NS, DF - 450, p['ani_W0'].shape[2]),
                                jnp.float32)], axis=1)

    def bs(a):
        nd = a.ndim
        return pl.BlockSpec(a.shape, lambda i, _n=nd: (0,) * _n)

    ins = [sorted_feats, offs, coul,
           w0, p['ani_W1'], p['ani_W2'], p['ani_W3'],
           p['ani_b0'][:, None, :], p['ani_b1'][:, None, :],
           p['ani_b2'][:, None, :], p['ani_b3'][:, None, :]]
    specs = [pl.BlockSpec((TB, DF), lambda i: (i, 0))]
    specs += [bs(a) for a in ins[1:]]

    out = pl.pallas_call(
        _body_b,
        grid=(NT // TB,),
        in_specs=specs,
        out_specs=pl.BlockSpec((1, NM), lambda i: (0, 0)),
        out_shape=jax.ShapeDtypeStruct((1, NM), jnp.float32),
    )(*ins)
    return out


def kernel(species, coordinates, net_charge, params):
    feats, coul, q2, rank_f, offs = _run_a(
        species, coordinates, net_charge, params)
    feats2d = feats.reshape(NT, DF)
    rank_i = rank_f.reshape(NT)
    offs16 = offs.reshape(16)
    sorted_feats = _sc_dispatch(feats2d, species.reshape(NT), rank_i, offs16)
    mol_e = _run_b(sorted_feats, offs, coul.reshape(1, NM), params)
    return species, mol_e.reshape(NM), q2.reshape(NM, NA)


# speedup vs baseline: 1.0855x; 1.0855x over previous
import functools

import jax
import jax.numpy as jnp
from jax import lax
from jax.experimental import pallas as pl
from jax.experimental.pallas import tpu as pltpu
from jax.experimental.pallas import tpu_sc as plsc

A0 = 0.529177249
SIG2 = [0.5515909**2, 1.8886297**2, 1.3225029**2, 1.2316629**2,
        2.1884933**2, 1.7750372**2, 1.3677907**2, 1.3820058**2]
NM, NA, NS = 128, 64, 8
G = 8
T = G * NA
STEPS = NM // G
NT = NM * NA
DF = 512
TB = 256
HP = jax.lax.Precision.HIGHEST


def _dot2(a, b):
    bh = b.astype(jnp.bfloat16).astype(jnp.float32)
    return jnp.dot(a, bh) + jnp.dot(a, b - bh)


def _dot2a(a, b):
    ah = a.astype(jnp.bfloat16).astype(jnp.float32)
    return jnp.dot(ah, b) + jnp.dot(a - ah, b)


def _celu(x):
    return jnp.maximum(x, 0.0) + 0.1 * (jnp.exp(jnp.minimum(x * 10.0, 0.0)) - 1.0)


def _softplus(x):
    return jnp.maximum(x, 0.0) + jnp.log(1.0 + jnp.exp(-jnp.abs(x)))


def _erf(x):
    t = 1.0 / (1.0 + 0.3275911 * x)
    p = t * (0.254829592 + t * (-0.284496736 + t * (1.421413741
              + t * (-1.453152027 + t * 1.061405429))))
    return 1.0 - p * jnp.exp(-x * x)


def _body_a(spc_ref, spr_ref, cc_ref, cr_ref, nq_ref, s2c_ref,
            w16a_ref, w16q_ref,
            c0a_ref, c0q_ref, c0qr_ref, c0er_ref, c1_ref, c2_ref, c3_ref,
            cb0_ref, cb1_ref, cb2_ref, cb3_ref,
            ft_ref, coul_ref, q_ref, rk_ref, cnt_ref,
            nbr_ref, jbd_ref, carry_ref):
    pid = pl.program_id(0)

    @pl.when(pid == 0)
    def _init():
        nbr_ref[...] = jnp.zeros((2, T // 2, T // 2), jnp.float32)
        jbd_ref[...] = jnp.zeros((2, T // 2, T // 2), jnp.float32)
        carry_ref[...] = jnp.zeros((1, NS), jnp.float32)

    sp_c = spc_ref[0]
    sp_r = spr_ref[0]
    onehot = (sp_c == jax.lax.broadcasted_iota(jnp.int32, (T, NS), 1)
              ).astype(jnp.float32)
    sig2_c = jnp.dot(onehot, s2c_ref[...])
    sig2_r = jnp.full((1, T), SIG2[0], jnp.float32)
    for e in range(1, NS):
        sig2_r = jnp.where(sp_r == e, jnp.float32(SIG2[e]), sig2_r)

    ii = jax.lax.broadcasted_iota(jnp.int32, (NA, NA), 0)
    jj = jax.lax.broadcasted_iota(jnp.int32, (NA, NA), 1)
    offm = jnp.where(ii == jj, 0.0, 1.0).astype(jnp.float32)

    for g in range(G):
        hf, og = divmod(g, G // 2)
        sl = pl.ds(g * NA, NA)
        sh = pl.ds(og * NA, NA)
        d2 = jnp.full((NA, NA), 1e-16, jnp.float32)
        for ax in range(3):
            col = cc_ref[0, sl, ax:ax + 1]
            row = cr_ref[0, ax:ax + 1, sl]
            dif = col - row
            d2 = d2 + dif * dif
        dist = jnp.sqrt(d2) * jnp.float32(1.0 / A0)
        nbr_ref[hf, sh, sh] = jnp.exp(-dist) * offm
        s2 = sig2_c[g * NA:(g + 1) * NA, :] + sig2_r[:, g * NA:(g + 1) * NA]
        x = dist * jax.lax.rsqrt(2.0 * s2)
        jbd_ref[hf, sh, sh] = _erf(x) / dist * offm

    ti = jax.lax.broadcasted_iota(jnp.int32, (T, T), 0)
    tj = jax.lax.broadcasted_iota(jnp.int32, (T, T), 1)
    stri = jnp.where(tj < ti, 1.0, 0.0).astype(jnp.float32)
    carry = carry_ref[...]
    rank_tot = jnp.dot(stri, onehot) + carry
    rk_ref[0] = jnp.sum(rank_tot * onehot, axis=1,
                        keepdims=True).astype(jnp.int32)
    carry_new = carry + jnp.sum(onehot, axis=0, keepdims=True)
    carry_ref[...] = carry_new
    oe = jax.lax.broadcasted_iota(jnp.int32, (NS, 16), 0)
    oj = jax.lax.broadcasted_iota(jnp.int32, (NS, 16), 1)
    mlt = jnp.where(oe < oj, 1.0, 0.0).astype(jnp.float32)
    cnt_ref[...] = _dot2a(carry_new, mlt).astype(jnp.int32)

    gi = jax.lax.broadcasted_iota(jnp.int32, (G, T), 0)
    gt = jax.lax.broadcasted_iota(jnp.int32, (G, T), 1)
    seg = jnp.where(gt // NA == gi, 1.0, 0.0).astype(jnp.float32)
    si = jax.lax.broadcasted_iota(jnp.int32, (T, G), 0)
    sj = jax.lax.broadcasted_iota(jnp.int32, (T, G), 1)
    segT = jnp.where(si // NA == sj, 1.0, 0.0).astype(jnp.float32)

    base16 = jnp.concatenate([cc_ref[0], onehot], axis=1)
    phi_aev = jnp.tanh(jnp.dot(base16, w16a_ref[...]))
    H = T // 2
    aev = jnp.tanh(jnp.concatenate(
        [jnp.dot(nbr_ref[0], phi_aev[:H]),
         jnp.dot(nbr_ref[1], phi_aev[H:])], axis=0))

    c1 = c1_ref[...]; c2 = c2_ref[...]; c3 = c3_ref[...]
    cb1 = cb1_ref[...]; cb2 = cb2_ref[...]; cb3 = cb3_ref[...]
    nq = nq_ref[0]

    def chi_tail(pre):
        h = _celu(pre)
        h = _celu(jnp.dot(h, c1) + cb1)
        h = _celu(jnp.dot(h, c2) + cb2)
        return _softplus(jnp.dot(h, c3) + cb3)

    def equil(chi):
        sums = _dot2(seg, chi)
        k_net = 1.0 + jnp.abs(nq) / sums
        k_p = jnp.where(nq > 0, k_net, 1.0)
        k_n = jnp.where(nq < 0, k_net, 1.0)
        ktab = jnp.concatenate(
            [k_n, k_p * (sums * jnp.float32(1.0 / NA))], axis=1)
        kexp = _dot2(segT, ktab)
        return -kexp[:, 0:1] * chi + kexp[:, 1:2]

    def esp_of(q):
        return jnp.concatenate(
            [jnp.dot(jbd_ref[0], q[:H]),
             jnp.dot(jbd_ref[1], q[H:])], axis=0)

    h_aev = jnp.dot(aev, c0a_ref[...]) + cb0_ref[...]

    chi1 = chi_tail(h_aev)
    q1 = equil(chi1)
    esp1 = esp_of(q1)

    phi_qr = jnp.tanh(jnp.dot(base16, w16q_ref[...]))
    wphi = q1 * phi_qr
    qraev = jnp.tanh(jnp.concatenate(
        [jnp.dot(nbr_ref[0], wphi[:H]),
         jnp.dot(nbr_ref[1], wphi[H:])], axis=0))

    pre2 = (h_aev + jnp.dot(qraev, c0q_ref[...])
            + q1 * c0qr_ref[...] + esp1 * c0er_ref[...])
    chi2 = chi_tail(pre2)
    q2 = equil(chi2)
    esp2 = esp_of(q2)

    molid = (G * pid + jax.lax.broadcasted_iota(jnp.int32, (T, 1), 0) // NA
             ).astype(jnp.float32)
    ft_ref[0] = jnp.concatenate(
        [aev, qraev, q2, esp2, molid,
         jnp.zeros((T, DF - 451), jnp.float32)], axis=1)

    coul_ref[0] = 0.5 * jnp.dot(seg, q2 * esp2)
    q_ref[0] = q2


def _run_a(species, coordinates, net_charge, params):
    sp_col = species.reshape(STEPS, T, 1)
    sp_row = species.reshape(STEPS, 1, T)
    cf = coordinates.reshape(STEPS, T, 3)
    coords_c = jnp.pad(cf, ((0, 0), (0, 0), (0, 5)))
    coords_r = jnp.pad(cf.transpose(0, 2, 1), ((0, 0), (0, 5), (0, 0)))
    netq = net_charge.reshape(STEPS, G, 1)
    sig2 = jnp.asarray(SIG2, jnp.float32).reshape(NS, 1)

    p = params
    w16a = jnp.concatenate(
        [jnp.pad(p['W_aev'][:3], ((0, 5), (0, 0))), p['W_aev'][3:]], axis=0)
    w16q = jnp.concatenate(
        [jnp.pad(p['W_qr'][:3], ((0, 5), (0, 0))), p['W_qr'][3:]], axis=0)
    c0 = p['chi_W0']
    c0a, c0q = c0[:384], c0[384:448]
    c0qr, c0er = c0[448:449], c0[449:450]

    def bs(a):
        nd = a.ndim
        return pl.BlockSpec(a.shape, lambda i, _n=nd: (0,) * _n)

    ins = [sp_col, sp_row, coords_c, coords_r, netq, sig2,
           w16a, w16q,
           c0a, c0q, c0qr, c0er, p['chi_W1'], p['chi_W2'], p['chi_W3'],
           p['chi_b0'].reshape(1, -1), p['chi_b1'].reshape(1, -1),
           p['chi_b2'].reshape(1, -1), p['chi_b3'].reshape(1, -1)]

    specs = [pl.BlockSpec((1, T, 1), lambda i: (i, 0, 0)),
             pl.BlockSpec((1, 1, T), lambda i: (i, 0, 0)),
             pl.BlockSpec((1, T, 8), lambda i: (i, 0, 0)),
             pl.BlockSpec((1, 8, T), lambda i: (i, 0, 0)),
             pl.BlockSpec((1, G, 1), lambda i: (i, 0, 0))]
    specs += [bs(a) for a in ins[5:]]

    out_shapes = (jax.ShapeDtypeStruct((STEPS, T, DF), jnp.float32),
                  jax.ShapeDtypeStruct((STEPS, G, 1), jnp.float32),
                  jax.ShapeDtypeStruct((STEPS, T, 1), jnp.float32),
                  jax.ShapeDtypeStruct((STEPS, T, 1), jnp.int32),
                  jax.ShapeDtypeStruct((1, 16), jnp.int32))
    out_specs = (pl.BlockSpec((1, T, DF), lambda i: (i, 0, 0)),
                 pl.BlockSpec((1, G, 1), lambda i: (i, 0, 0)),
                 pl.BlockSpec((1, T, 1), lambda i: (i, 0, 0)),
                 pl.BlockSpec((1, T, 1), lambda i: (i, 0, 0)),
                 pl.BlockSpec((1, 16), lambda i: (0, 0)))

    return pl.pallas_call(
        _body_a,
        grid=(STEPS,),
        in_specs=specs,
        out_specs=out_specs,
        out_shape=out_shapes,
        scratch_shapes=[pltpu.VMEM((2, T // 2, T // 2), jnp.float32),
                        pltpu.VMEM((2, T // 2, T // 2), jnp.float32),
                        pltpu.VMEM((1, NS), jnp.float32)],
    )(*ins)


def _sc_dispatch(feats, species_flat, rank_i, offs16):
    mesh = plsc.VectorSubcoreMesh(core_axis_name="c", subcore_axis_name="s")
    NB, RB = 4, 64

    @functools.partial(
        pl.kernel, mesh=mesh,
        out_type=jax.ShapeDtypeStruct((NT, DF), jnp.float32),
        compiler_params=pltpu.CompilerParams(needs_layout_passes=False),
        scratch_types=[
            pltpu.VMEM((TB,), jnp.int32),
            pltpu.VMEM((TB,), jnp.int32),
            pltpu.VMEM((16,), jnp.int32),
            pltpu.VMEM((NB, RB), jnp.int32),
            pltpu.VMEM((3, RB, DF), jnp.float32),
            pltpu.SemaphoreType.DMA,
            pltpu.SemaphoreType.DMA,
        ])
    def k(ft_hbm, sp_hbm, rk_hbm, off_hbm, out_hbm,
          sp_v, rk_v, off_v, dest_v, rows_v, sem_i, sem_o):
        wid = lax.axis_index("s") * 2 + lax.axis_index("c")
        base = wid * TB
        ih, oh = {}, {}

        def start_in(b):
            ih[b] = pltpu.async_copy(
                ft_hbm.at[pl.ds(base + b * RB, RB)], rows_v.at[b % 3], sem_i)

        for b in range(3):
            start_in(b)
        pltpu.sync_copy(sp_hbm.at[pl.ds(base, TB)], sp_v)
        pltpu.sync_copy(rk_hbm.at[pl.ds(base, TB)], rk_v)
        pltpu.sync_copy(off_hbm, off_v)
        for j in range(16):
            s = sp_v[pl.ds(j * 16, 16)]
            r = rk_v[pl.ds(j * 16, 16)]
            o = plsc.load_gather(off_v, [s])
            dest_v[j // 4, pl.ds((j % 4) * 16, 16)] = o + r
        for b in range(NB):
            ih[b].wait()
            oh[b] = pltpu.async_copy(
                rows_v.at[b % 3], out_hbm.at[dest_v.at[b]], sem_o)
            if b + 3 < NB:
                oh[b].wait()
                start_in(b + 3)
        for b in range(NB):
            if not (b + 3 < NB):
                oh[b].wait()

    return k(feats, species_flat, rank_i, offs16)


def _body_b(sf_ref, cnt_ref, coul_ref,
            w0_ref, w1_ref, w2_ref, w3_ref,
            b0_ref, b1_ref, b2_ref, b3_ref,
            out_ref):
    b = pl.program_id(0)

    @pl.when(b == 0)
    def _init():
        out_ref[...] = coul_ref[...]

    sf = sf_ref[...]
    molid = sf[:, 450:451]

    offs = cnt_ref[...].astype(jnp.float32)[:, 0:NS + 1]
    glob = (TB * b + jax.lax.broadcasted_iota(jnp.int32, (TB, 1), 0)
            ).astype(jnp.float32)
    lo_f = jnp.float32(TB) * b.astype(jnp.float32)
    hi_f = lo_f + jnp.float32(TB - 1)
    off8 = offs[:, 0:NS]
    e_lo = (jnp.sum(jnp.where(off8 <= lo_f, 1.0, 0.0)) - 1.0).astype(jnp.int32)
    e_hi = (jnp.sum(jnp.where(off8 <= hi_f, 1.0, 0.0)) - 1.0).astype(jnp.int32)

    def expert(i, acc):
        e = e_lo + i
        seg_lo = jnp.sum(jnp.where(
            jax.lax.broadcasted_iota(jnp.int32, (1, NS + 1), 1) == e,
            offs, 0.0))
        seg_hi = jnp.sum(jnp.where(
            jax.lax.broadcasted_iota(jnp.int32, (1, NS + 1), 1) == e + 1,
            offs, 0.0))
        msk = jnp.logical_and(glob >= seg_lo, glob < seg_hi)
        h = _celu(jnp.dot(sf, w0_ref[e]) + b0_ref[e])
        h = _celu(jnp.dot(h, w1_ref[e]) + b1_ref[e])
        h = _celu(jnp.dot(h, w2_ref[e]) + b2_ref[e])
        o = jnp.dot(h, w3_ref[e]) + b3_ref[e]
        return acc + jnp.where(msk, o, 0.0)

    en = jax.lax.fori_loop(0, e_hi - e_lo + 1, expert,
                           jnp.zeros((TB, 1), jnp.float32))
    mi = jax.lax.broadcasted_iota(jnp.int32, (TB, NM), 1).astype(jnp.float32)
    oh = jnp.where(molid == mi, 1.0, 0.0).astype(jnp.float32)
    out_ref[...] += jnp.sum(en * oh, axis=0, keepdims=True)


def _run_b(sorted_feats, offs, coul, params):
    p = params
    w0 = jnp.concatenate(
        [p['ani_W0'], jnp.zeros((NS, DF - 450, p['ani_W0'].shape[2]),
                                jnp.float32)], axis=1)

    def bs(a):
        nd = a.ndim
        return pl.BlockSpec(a.shape, lambda i, _n=nd: (0,) * _n)

    ins = [sorted_feats, offs, coul,
           w0, p['ani_W1'], p['ani_W2'], p['ani_W3'],
           p['ani_b0'][:, None, :], p['ani_b1'][:, None, :],
           p['ani_b2'][:, None, :], p['ani_b3'][:, None, :]]
    specs = [pl.BlockSpec((TB, DF), lambda i: (i, 0))]
    specs += [bs(a) for a in ins[1:]]

    out = pl.pallas_call(
        _body_b,
        grid=(NT // TB,),
        in_specs=specs,
        out_specs=pl.BlockSpec((1, NM), lambda i: (0, 0)),
        out_shape=jax.ShapeDtypeStruct((1, NM), jnp.float32),
    )(*ins)
    return out


def kernel(species, coordinates, net_charge, params):
    feats, coul, q2, rank_f, offs = _run_a(
        species, coordinates, net_charge, params)
    feats2d = feats.reshape(NT, DF)
    rank_i = rank_f.reshape(NT)
    offs16 = offs.reshape(16)
    sorted_feats = _sc_dispatch(feats2d, species.reshape(NT), rank_i, offs16)
    mol_e = _run_b(sorted_feats, offs, coul.reshape(1, NM), params)
    return species, mol_e.reshape(NM), q2.reshape(NM, NA)


# speedup vs baseline: 1.1196x; 1.0314x over previous
import functools

import jax
import jax.numpy as jnp
from jax import lax
from jax.experimental import pallas as pl
from jax.experimental.pallas import tpu as pltpu
from jax.experimental.pallas import tpu_sc as plsc

A0 = 0.529177249
SIG2 = [0.5515909**2, 1.8886297**2, 1.3225029**2, 1.2316629**2,
        2.1884933**2, 1.7750372**2, 1.3677907**2, 1.3820058**2]
NM, NA, NS = 128, 64, 8
G = 8
T = G * NA
STEPS = NM // G
NT = NM * NA
DF = 512
TB = 256
HP = jax.lax.Precision.HIGHEST


def _dot2(a, b):
    bh = b.astype(jnp.bfloat16).astype(jnp.float32)
    return jnp.dot(a, bh) + jnp.dot(a, b - bh)


def _dot2a(a, b):
    ah = a.astype(jnp.bfloat16).astype(jnp.float32)
    return jnp.dot(ah, b) + jnp.dot(a - ah, b)


def _celu(x):
    return jnp.maximum(x, 0.0) + 0.1 * (jnp.exp(jnp.minimum(x * 10.0, 0.0)) - 1.0)


def _softplus(x):
    return jnp.maximum(x, 0.0) + jnp.log(1.0 + jnp.exp(-jnp.abs(x)))


def _erf(x):
    t = 1.0 / (1.0 + 0.3275911 * x)
    p = t * (0.254829592 + t * (-0.284496736 + t * (1.421413741
              + t * (-1.453152027 + t * 1.061405429))))
    return 1.0 - p * jnp.exp(-x * x)


def _body_a(spc_ref, spr_ref, cc_ref, cr_ref, nq_ref, s2c_ref,
            w16a_ref, w16q_ref,
            c0a_ref, c0q_ref, c0qr_ref, c0er_ref, c1_ref, c2_ref, c3_ref,
            cb0_ref, cb1_ref, cb2_ref, cb3_ref,
            ft_ref, coul_ref, q_ref, rk_ref, cnt_ref,
            nbr_ref, jbd_ref, carry_ref):
    pid = pl.program_id(0)

    @pl.when(pid == 0)
    def _init():
        nbr_ref[...] = jnp.zeros((2, T // 2, T // 2), jnp.float32)
        jbd_ref[...] = jnp.zeros((2, T // 2, T // 2), jnp.float32)
        carry_ref[...] = jnp.zeros((1, NS), jnp.float32)

    sp_c = spc_ref[0]
    sp_r = spr_ref[0]
    onehot = (sp_c == jax.lax.broadcasted_iota(jnp.int32, (T, NS), 1)
              ).astype(jnp.float32)
    sig2_c = jnp.dot(onehot, s2c_ref[...])
    sig2_r = jnp.full((1, T), SIG2[0], jnp.float32)
    for e in range(1, NS):
        sig2_r = jnp.where(sp_r == e, jnp.float32(SIG2[e]), sig2_r)

    ii = jax.lax.broadcasted_iota(jnp.int32, (NA, NA), 0)
    jj = jax.lax.broadcasted_iota(jnp.int32, (NA, NA), 1)
    offm = jnp.where(ii == jj, 0.0, 1.0).astype(jnp.float32)

    for g in range(G):
        hf, og = divmod(g, G // 2)
        sl = pl.ds(g * NA, NA)
        sh = pl.ds(og * NA, NA)
        d2 = jnp.full((NA, NA), 1e-16, jnp.float32)
        for ax in range(3):
            col = cc_ref[0, sl, ax:ax + 1]
            row = cr_ref[0, ax:ax + 1, sl]
            dif = col - row
            d2 = d2 + dif * dif
        dist = jnp.sqrt(d2) * jnp.float32(1.0 / A0)
        nbr_ref[hf, sh, sh] = jnp.exp(-dist) * offm
        s2 = sig2_c[g * NA:(g + 1) * NA, :] + sig2_r[:, g * NA:(g + 1) * NA]
        x = dist * jax.lax.rsqrt(2.0 * s2)
        jbd_ref[hf, sh, sh] = _erf(x) / dist * offm

    ti = jax.lax.broadcasted_iota(jnp.int32, (T, T), 0)
    tj = jax.lax.broadcasted_iota(jnp.int32, (T, T), 1)
    stri = jnp.where(tj < ti, 1.0, 0.0).astype(jnp.float32)
    carry = carry_ref[...]
    rank_tot = jnp.dot(stri, onehot) + carry
    rk_ref[0] = jnp.sum(rank_tot * onehot, axis=1,
                        keepdims=True).astype(jnp.int32)
    carry_new = carry + jnp.sum(onehot, axis=0, keepdims=True)
    carry_ref[...] = carry_new
    oe = jax.lax.broadcasted_iota(jnp.int32, (NS, 16), 0)
    oj = jax.lax.broadcasted_iota(jnp.int32, (NS, 16), 1)
    mlt = jnp.where(oe < oj, 1.0, 0.0).astype(jnp.float32)
    cnt_ref[...] = _dot2a(carry_new, mlt).astype(jnp.int32)

    gi = jax.lax.broadcasted_iota(jnp.int32, (G, T), 0)
    gt = jax.lax.broadcasted_iota(jnp.int32, (G, T), 1)
    seg = jnp.where(gt // NA == gi, 1.0, 0.0).astype(jnp.float32)
    si = jax.lax.broadcasted_iota(jnp.int32, (T, G), 0)
    sj = jax.lax.broadcasted_iota(jnp.int32, (T, G), 1)
    segT = jnp.where(si // NA == sj, 1.0, 0.0).astype(jnp.float32)

    base16 = jnp.concatenate([cc_ref[0], onehot], axis=1)
    phi_aev = jnp.tanh(jnp.dot(base16, w16a_ref[...]))
    H = T // 2
    aev = jnp.tanh(jnp.concatenate(
        [jnp.dot(nbr_ref[0], phi_aev[:H]),
         jnp.dot(nbr_ref[1], phi_aev[H:])], axis=0))

    c1 = c1_ref[...]; c2 = c2_ref[...]; c3 = c3_ref[...]
    cb1 = cb1_ref[...]; cb2 = cb2_ref[...]; cb3 = cb3_ref[...]
    nq = nq_ref[0]

    def chi_tail(pre):
        h = _celu(pre)
        h = _celu(jnp.dot(h, c1) + cb1)
        h = _celu(jnp.dot(h, c2) + cb2)
        return _softplus(jnp.dot(h, c3) + cb3)

    def equil(chi):
        sums = _dot2(seg, chi)
        k_net = 1.0 + jnp.abs(nq) / sums
        k_p = jnp.where(nq > 0, k_net, 1.0)
        k_n = jnp.where(nq < 0, k_net, 1.0)
        ktab = jnp.concatenate(
            [k_n, k_p * (sums * jnp.float32(1.0 / NA))], axis=1)
        kexp = _dot2(segT, ktab)
        return -kexp[:, 0:1] * chi + kexp[:, 1:2]

    def esp_of(q):
        return jnp.concatenate(
            [jnp.dot(jbd_ref[0], q[:H]),
             jnp.dot(jbd_ref[1], q[H:])], axis=0)

    h_aev = jnp.dot(aev, c0a_ref[...]) + cb0_ref[...]

    chi1 = chi_tail(h_aev)
    q1 = equil(chi1)
    esp1 = esp_of(q1)

    phi_qr = jnp.tanh(jnp.dot(base16, w16q_ref[...]))
    wphi = q1 * phi_qr
    qraev = jnp.tanh(jnp.concatenate(
        [jnp.dot(nbr_ref[0], wphi[:H]),
         jnp.dot(nbr_ref[1], wphi[H:])], axis=0))

    pre2 = (h_aev + jnp.dot(qraev, c0q_ref[...])
            + q1 * c0qr_ref[...] + esp1 * c0er_ref[...])
    chi2 = chi_tail(pre2)
    q2 = equil(chi2)
    esp2 = esp_of(q2)

    molid = (G * pid + jax.lax.broadcasted_iota(jnp.int32, (T, 1), 0) // NA
             ).astype(jnp.float32)
    ft = jnp.concatenate(
        [aev, qraev, q2, esp2, molid,
         jnp.zeros((T, DF - 451), jnp.float32)], axis=1)
    lo_b = jax.lax.bitcast_convert_type(
        ft[:, :256].astype(jnp.bfloat16).astype(jnp.float32), jnp.int32)
    hi_b = jax.lax.bitcast_convert_type(
        ft[:, 256:].astype(jnp.bfloat16).astype(jnp.float32), jnp.int32)
    ft_ref[0] = ((lo_b & jnp.int32(-65536))
                 | jax.lax.shift_right_logical(hi_b, 16))

    coul_ref[0] = 0.5 * jnp.dot(seg, q2 * esp2)
    q_ref[0] = q2


def _run_a(species, coordinates, net_charge, params):
    sp_col = species.reshape(STEPS, T, 1)
    sp_row = species.reshape(STEPS, 1, T)
    cf = coordinates.reshape(STEPS, T, 3)
    coords_c = jnp.pad(cf, ((0, 0), (0, 0), (0, 5)))
    coords_r = jnp.pad(cf.transpose(0, 2, 1), ((0, 0), (0, 5), (0, 0)))
    netq = net_charge.reshape(STEPS, G, 1)
    sig2 = jnp.asarray(SIG2, jnp.float32).reshape(NS, 1)

    p = params
    w16a = jnp.concatenate(
        [jnp.pad(p['W_aev'][:3], ((0, 5), (0, 0))), p['W_aev'][3:]], axis=0)
    w16q = jnp.concatenate(
        [jnp.pad(p['W_qr'][:3], ((0, 5), (0, 0))), p['W_qr'][3:]], axis=0)
    c0 = p['chi_W0']
    c0a, c0q = c0[:384], c0[384:448]
    c0qr, c0er = c0[448:449], c0[449:450]

    def bs(a):
        nd = a.ndim
        return pl.BlockSpec(a.shape, lambda i, _n=nd: (0,) * _n)

    ins = [sp_col, sp_row, coords_c, coords_r, netq, sig2,
           w16a, w16q,
           c0a, c0q, c0qr, c0er, p['chi_W1'], p['chi_W2'], p['chi_W3'],
           p['chi_b0'].reshape(1, -1), p['chi_b1'].reshape(1, -1),
           p['chi_b2'].reshape(1, -1), p['chi_b3'].reshape(1, -1)]

    specs = [pl.BlockSpec((1, T, 1), lambda i: (i, 0, 0)),
             pl.BlockSpec((1, 1, T), lambda i: (i, 0, 0)),
             pl.BlockSpec((1, T, 8), lambda i: (i, 0, 0)),
             pl.BlockSpec((1, 8, T), lambda i: (i, 0, 0)),
             pl.BlockSpec((1, G, 1), lambda i: (i, 0, 0))]
    specs += [bs(a) for a in ins[5:]]

    out_shapes = (jax.ShapeDtypeStruct((STEPS, T, 256), jnp.int32),
                  jax.ShapeDtypeStruct((STEPS, G, 1), jnp.float32),
                  jax.ShapeDtypeStruct((STEPS, T, 1), jnp.float32),
                  jax.ShapeDtypeStruct((STEPS, T, 1), jnp.int32),
                  jax.ShapeDtypeStruct((1, 16), jnp.int32))
    out_specs = (pl.BlockSpec((1, T, 256), lambda i: (i, 0, 0)),
                 pl.BlockSpec((1, G, 1), lambda i: (i, 0, 0)),
                 pl.BlockSpec((1, T, 1), lambda i: (i, 0, 0)),
                 pl.BlockSpec((1, T, 1), lambda i: (i, 0, 0)),
                 pl.BlockSpec((1, 16), lambda i: (0, 0)))

    return pl.pallas_call(
        _body_a,
        grid=(STEPS,),
        in_specs=specs,
        out_specs=out_specs,
        out_shape=out_shapes,
        scratch_shapes=[pltpu.VMEM((2, T // 2, T // 2), jnp.float32),
                        pltpu.VMEM((2, T // 2, T // 2), jnp.float32),
                        pltpu.VMEM((1, NS), jnp.float32)],
    )(*ins)


def _sc_dispatch(feats, species_flat, rank_i, offs16):
    mesh = plsc.VectorSubcoreMesh(core_axis_name="c", subcore_axis_name="s")
    NB, RB = 4, 64

    @functools.partial(
        pl.kernel, mesh=mesh,
        out_type=jax.ShapeDtypeStruct((NT, 256), jnp.int32),
        compiler_params=pltpu.CompilerParams(needs_layout_passes=False),
        scratch_types=[
            pltpu.VMEM((TB,), jnp.int32),
            pltpu.VMEM((TB,), jnp.int32),
            pltpu.VMEM((16,), jnp.int32),
            pltpu.VMEM((NB, RB), jnp.int32),
            pltpu.VMEM((3, RB, 256), jnp.int32),
            pltpu.SemaphoreType.DMA,
            pltpu.SemaphoreType.DMA,
        ])
    def k(ft_hbm, sp_hbm, rk_hbm, off_hbm, out_hbm,
          sp_v, rk_v, off_v, dest_v, rows_v, sem_i, sem_o):
        wid = lax.axis_index("s") * 2 + lax.axis_index("c")
        base = wid * TB
        ih, oh = {}, {}

        def start_in(b):
            ih[b] = pltpu.async_copy(
                ft_hbm.at[pl.ds(base + b * RB, RB)], rows_v.at[b % 3], sem_i)

        for b in range(3):
            start_in(b)
        pltpu.sync_copy(sp_hbm.at[pl.ds(base, TB)], sp_v)
        pltpu.sync_copy(rk_hbm.at[pl.ds(base, TB)], rk_v)
        pltpu.sync_copy(off_hbm, off_v)
        for j in range(16):
            s = sp_v[pl.ds(j * 16, 16)]
            r = rk_v[pl.ds(j * 16, 16)]
            o = plsc.load_gather(off_v, [s])
            dest_v[j // 4, pl.ds((j % 4) * 16, 16)] = o + r
        for b in range(NB):
            ih[b].wait()
            oh[b] = pltpu.async_copy(
                rows_v.at[b % 3], out_hbm.at[dest_v.at[b]], sem_o)
            if b + 3 < NB:
                oh[b].wait()
                start_in(b + 3)
        for b in range(NB):
            if not (b + 3 < NB):
                oh[b].wait()

    return k(feats, species_flat, rank_i, offs16)


def _body_b(sf_ref, cnt_ref, coul_ref,
            w0_ref, w1_ref, w2_ref, w3_ref,
            b0_ref, b1_ref, b2_ref, b3_ref,
            out_ref):
    b = pl.program_id(0)

    @pl.when(b == 0)
    def _init():
        out_ref[...] = coul_ref[...]

    pk = sf_ref[...]
    f_lo = jax.lax.bitcast_convert_type(
        pk & jnp.int32(-65536), jnp.float32)
    f_hi = jax.lax.bitcast_convert_type(
        jax.lax.shift_left(pk, 16), jnp.float32)
    molid = f_hi[:, 194:195]

    offs = cnt_ref[...].astype(jnp.float32)[:, 0:NS + 1]
    glob = (TB * b + jax.lax.broadcasted_iota(jnp.int32, (TB, 1), 0)
            ).astype(jnp.float32)
    lo_f = jnp.float32(TB) * b.astype(jnp.float32)
    hi_f = lo_f + jnp.float32(TB - 1)
    off8 = offs[:, 0:NS]
    e_lo = (jnp.sum(jnp.where(off8 <= lo_f, 1.0, 0.0)) - 1.0).astype(jnp.int32)
    e_hi = (jnp.sum(jnp.where(off8 <= hi_f, 1.0, 0.0)) - 1.0).astype(jnp.int32)

    def expert(i, acc):
        e = e_lo + i
        seg_lo = jnp.sum(jnp.where(
            jax.lax.broadcasted_iota(jnp.int32, (1, NS + 1), 1) == e,
            offs, 0.0))
        seg_hi = jnp.sum(jnp.where(
            jax.lax.broadcasted_iota(jnp.int32, (1, NS + 1), 1) == e + 1,
            offs, 0.0))
        msk = jnp.logical_and(glob >= seg_lo, glob < seg_hi)
        h = _celu(jnp.dot(f_lo, w0_ref[e, pl.ds(0, 256)])
                  + jnp.dot(f_hi, w0_ref[e, pl.ds(256, 256)]) + b0_ref[e])
        h = _celu(jnp.dot(h, w1_ref[e]) + b1_ref[e])
        h = _celu(jnp.dot(h, w2_ref[e]) + b2_ref[e])
        o = jnp.dot(h, w3_ref[e]) + b3_ref[e]
        return acc + jnp.where(msk, o, 0.0)

    en = jax.lax.fori_loop(0, e_hi - e_lo + 1, expert,
                           jnp.zeros((TB, 1), jnp.float32))
    mi = jax.lax.broadcasted_iota(jnp.int32, (TB, NM), 1).astype(jnp.float32)
    oh = jnp.where(molid == mi, 1.0, 0.0).astype(jnp.float32)
    out_ref[...] += jnp.sum(en * oh, axis=0, keepdims=True)


def _run_b(sorted_feats, offs, coul, params):
    p = params
    w0 = jnp.concatenate(
        [p['ani_W0'], jnp.zeros((NS, DF - 450, p['ani_W0'].shape[2]),
                                jnp.float32)], axis=1)

    def bs(a):
        nd = a.ndim
        return pl.BlockSpec(a.shape, lambda i, _n=nd: (0,) * _n)

    ins = [sorted_feats, offs, coul,
           w0, p['ani_W1'], p['ani_W2'], p['ani_W3'],
           p['ani_b0'][:, None, :], p['ani_b1'][:, None, :],
           p['ani_b2'][:, None, :], p['ani_b3'][:, None, :]]
    specs = [pl.BlockSpec((TB, 256), lambda i: (i, 0))]
    specs += [bs(a) for a in ins[1:]]

    out = pl.pallas_call(
        _body_b,
        grid=(NT // TB,),
        in_specs=specs,
        out_specs=pl.BlockSpec((1, NM), lambda i: (0, 0)),
        out_shape=jax.ShapeDtypeStruct((1, NM), jnp.float32),
    )(*ins)
    return out


def kernel(species, coordinates, net_charge, params):
    feats, coul, q2, rank_f, offs = _run_a(
        species, coordinates, net_charge, params)
    feats2d = feats.reshape(NT, 256)
    rank_i = rank_f.reshape(NT)
    offs16 = offs.reshape(16)
    sorted_feats = _sc_dispatch(feats2d, species.reshape(NT), rank_i, offs16)
    mol_e = _run_b(sorted_feats, offs, coul.reshape(1, NM), params)
    return species, mol_e.reshape(NM), q2.reshape(NM, NA)


# speedup vs baseline: 1.1226x; 1.0026x over previous
import functools

import jax
import jax.numpy as jnp
from jax import lax
from jax.experimental import pallas as pl
from jax.experimental.pallas import tpu as pltpu
from jax.experimental.pallas import tpu_sc as plsc

A0 = 0.529177249
SIG2 = [0.5515909**2, 1.8886297**2, 1.3225029**2, 1.2316629**2,
        2.1884933**2, 1.7750372**2, 1.3677907**2, 1.3820058**2]
NM, NA, NS = 128, 64, 8
G = 8
T = G * NA
STEPS = NM // G
NT = NM * NA
DF = 512
TB = 256


def _dot2(a, b):
    bh = b.astype(jnp.bfloat16).astype(jnp.float32)
    return jnp.dot(a, bh) + jnp.dot(a, b - bh)


def _dot2a(a, b):
    ah = a.astype(jnp.bfloat16).astype(jnp.float32)
    return jnp.dot(ah, b) + jnp.dot(a - ah, b)


def _celu(x):
    return jnp.maximum(x, 0.0) + 0.1 * (jnp.exp(jnp.minimum(x * 10.0, 0.0)) - 1.0)


def _softplus(x):
    return jnp.maximum(x, 0.0) + jnp.log(1.0 + jnp.exp(-jnp.abs(x)))


def _erf(x):
    t = 1.0 / (1.0 + 0.3275911 * x)
    p = t * (0.254829592 + t * (-0.284496736 + t * (1.421413741
              + t * (-1.453152027 + t * 1.061405429))))
    return 1.0 - p * jnp.exp(-x * x)


def _body_a(spc_ref, spr_ref, cc_ref, cr_ref, nq_ref, s2c_ref,
            w16a_ref, w16q_ref,
            c0a_ref, c0q_ref, c0qr_ref, c0er_ref, c1_ref, c2_ref, c3_ref,
            cb0_ref, cb1_ref, cb2_ref, cb3_ref,
            ft_ref, coul_ref, q_ref, rk_ref, cnt_ref,
            nbr_ref, jbd_ref, carry_ref):
    pid = pl.program_id(0)

    @pl.when(pid == 0)
    def _init():
        nbr_ref[...] = jnp.zeros((2, T // 2, T // 2), jnp.float32)
        jbd_ref[...] = jnp.zeros((2, T // 2, T // 2), jnp.float32)
        carry_ref[...] = jnp.zeros((1, NS), jnp.float32)

    sp_c = spc_ref[0]
    sp_r = spr_ref[0]
    onehot = (sp_c == jax.lax.broadcasted_iota(jnp.int32, (T, NS), 1)
              ).astype(jnp.float32)
    sig2_c = jnp.dot(onehot, s2c_ref[...])
    sig2_r = jnp.full((1, T), SIG2[0], jnp.float32)
    for e in range(1, NS):
        sig2_r = jnp.where(sp_r == e, jnp.float32(SIG2[e]), sig2_r)

    ii = jax.lax.broadcasted_iota(jnp.int32, (NA, NA), 0)
    jj = jax.lax.broadcasted_iota(jnp.int32, (NA, NA), 1)
    offm = jnp.where(ii == jj, 0.0, 1.0).astype(jnp.float32)

    for g in range(G):
        hf, og = divmod(g, G // 2)
        sl = pl.ds(g * NA, NA)
        sh = pl.ds(og * NA, NA)
        d2 = jnp.full((NA, NA), 1e-16, jnp.float32)
        for ax in range(3):
            col = cc_ref[0, sl, ax:ax + 1]
            row = cr_ref[0, ax:ax + 1, sl]
            dif = col - row
            d2 = d2 + dif * dif
        dist = jnp.sqrt(d2) * jnp.float32(1.0 / A0)
        nbr_ref[hf, sh, sh] = jnp.exp(-dist) * offm
        s2 = sig2_c[g * NA:(g + 1) * NA, :] + sig2_r[:, g * NA:(g + 1) * NA]
        x = dist * jax.lax.rsqrt(2.0 * s2)
        jbd_ref[hf, sh, sh] = _erf(x) / dist * offm

    ti = jax.lax.broadcasted_iota(jnp.int32, (T, T), 0)
    tj = jax.lax.broadcasted_iota(jnp.int32, (T, T), 1)
    stri = jnp.where(tj < ti, 1.0, 0.0).astype(jnp.float32)
    carry = carry_ref[...]
    rank_tot = jnp.dot(stri, onehot) + carry
    rk_ref[0] = jnp.sum(rank_tot * onehot, axis=1,
                        keepdims=True).astype(jnp.int32)
    carry_new = carry + jnp.sum(onehot, axis=0, keepdims=True)
    carry_ref[...] = carry_new
    oe = jax.lax.broadcasted_iota(jnp.int32, (NS, 16), 0)
    oj = jax.lax.broadcasted_iota(jnp.int32, (NS, 16), 1)
    mlt = jnp.where(oe < oj, 1.0, 0.0).astype(jnp.float32)
    cnt_ref[...] = _dot2a(carry_new, mlt).astype(jnp.int32)

    gi = jax.lax.broadcasted_iota(jnp.int32, (G, T), 0)
    gt = jax.lax.broadcasted_iota(jnp.int32, (G, T), 1)
    seg = jnp.where(gt // NA == gi, 1.0, 0.0).astype(jnp.float32)
    si = jax.lax.broadcasted_iota(jnp.int32, (T, G), 0)
    sj = jax.lax.broadcasted_iota(jnp.int32, (T, G), 1)
    segT = jnp.where(si // NA == sj, 1.0, 0.0).astype(jnp.float32)

    base16 = jnp.concatenate([cc_ref[0], onehot], axis=1)
    phi_aev = jnp.tanh(jnp.dot(base16, w16a_ref[...]))
    H = T // 2
    aev = jnp.tanh(jnp.concatenate(
        [jnp.dot(nbr_ref[0], phi_aev[:H]),
         jnp.dot(nbr_ref[1], phi_aev[H:])], axis=0))

    c1 = c1_ref[...]; c2 = c2_ref[...]; c3 = c3_ref[...]
    cb1 = cb1_ref[...]; cb2 = cb2_ref[...]; cb3 = cb3_ref[...]
    nq = nq_ref[0]

    def chi_tail(pre):
        h = _celu(pre)
        h = _celu(jnp.dot(h, c1) + cb1)
        h = _celu(jnp.dot(h, c2) + cb2)
        return _softplus(jnp.dot(h, c3) + cb3)

    def equil(chi):
        sums = _dot2(seg, chi)
        k_net = 1.0 + jnp.abs(nq) / sums
        k_p = jnp.where(nq > 0, k_net, 1.0)
        k_n = jnp.where(nq < 0, k_net, 1.0)
        ktab = jnp.concatenate(
            [k_n, k_p * (sums * jnp.float32(1.0 / NA))], axis=1)
        kexp = _dot2(segT, ktab)
        return -kexp[:, 0:1] * chi + kexp[:, 1:2]

    def esp_of(q):
        return jnp.concatenate(
            [jnp.dot(jbd_ref[0], q[:H]),
             jnp.dot(jbd_ref[1], q[H:])], axis=0)

    h_aev = jnp.dot(aev, c0a_ref[...]) + cb0_ref[...]

    chi1 = chi_tail(h_aev)
    q1 = equil(chi1)
    esp1 = esp_of(q1)

    phi_qr = jnp.tanh(jnp.dot(base16, w16q_ref[...]))
    wphi = q1 * phi_qr
    qraev = jnp.tanh(jnp.concatenate(
        [jnp.dot(nbr_ref[0], wphi[:H]),
         jnp.dot(nbr_ref[1], wphi[H:])], axis=0))

    pre2 = (h_aev + jnp.dot(qraev, c0q_ref[...])
            + q1 * c0qr_ref[...] + esp1 * c0er_ref[...])
    chi2 = chi_tail(pre2)
    q2 = equil(chi2)
    esp2 = esp_of(q2)

    molid = (G * pid + jax.lax.broadcasted_iota(jnp.int32, (T, 1), 0) // NA
             ).astype(jnp.float32)
    ft = jnp.concatenate(
        [aev, qraev, q2, esp2, molid,
         jnp.zeros((T, DF - 451), jnp.float32)], axis=1)
    lo_b = jax.lax.bitcast_convert_type(
        ft[:, :256].astype(jnp.bfloat16).astype(jnp.float32), jnp.int32)
    hi_b = jax.lax.bitcast_convert_type(
        ft[:, 256:].astype(jnp.bfloat16).astype(jnp.float32), jnp.int32)
    ft_ref[0] = ((lo_b & jnp.int32(-65536))
                 | jax.lax.shift_right_logical(hi_b, 16))

    coul_ref[0] = 0.5 * jnp.dot(seg, q2 * esp2)
    q_ref[0] = q2


def _run_a(species, coordinates, net_charge, params):
    sp_col = species.reshape(STEPS, T, 1)
    sp_row = species.reshape(STEPS, 1, T)
    cf = coordinates.reshape(STEPS, T, 3)
    coords_c = jnp.pad(cf, ((0, 0), (0, 0), (0, 5)))
    coords_r = jnp.pad(cf.transpose(0, 2, 1), ((0, 0), (0, 5), (0, 0)))
    netq = net_charge.reshape(STEPS, G, 1)
    sig2 = jnp.asarray(SIG2, jnp.float32).reshape(NS, 1)

    p = params
    w16a = jnp.concatenate(
        [jnp.pad(p['W_aev'][:3], ((0, 5), (0, 0))), p['W_aev'][3:]], axis=0)
    w16q = jnp.concatenate(
        [jnp.pad(p['W_qr'][:3], ((0, 5), (0, 0))), p['W_qr'][3:]], axis=0)
    c0 = p['chi_W0']
    c0a, c0q = c0[:384], c0[384:448]
    c0qr, c0er = c0[448:449], c0[449:450]

    def bs(a):
        nd = a.ndim
        return pl.BlockSpec(a.shape, lambda i, _n=nd: (0,) * _n)

    ins = [sp_col, sp_row, coords_c, coords_r, netq, sig2,
           w16a, w16q,
           c0a, c0q, c0qr, c0er, p['chi_W1'], p['chi_W2'], p['chi_W3'],
           p['chi_b0'].reshape(1, -1), p['chi_b1'].reshape(1, -1),
           p['chi_b2'].reshape(1, -1), p['chi_b3'].reshape(1, -1)]

    specs = [pl.BlockSpec((1, T, 1), lambda i: (i, 0, 0)),
             pl.BlockSpec((1, 1, T), lambda i: (i, 0, 0)),
             pl.BlockSpec((1, T, 8), lambda i: (i, 0, 0)),
             pl.BlockSpec((1, 8, T), lambda i: (i, 0, 0)),
             pl.BlockSpec((1, G, 1), lambda i: (i, 0, 0))]
    specs += [bs(a) for a in ins[5:]]

    out_shapes = (jax.ShapeDtypeStruct((STEPS, T, 256), jnp.int32),
                  jax.ShapeDtypeStruct((STEPS, G, 1), jnp.float32),
                  jax.ShapeDtypeStruct((STEPS, T, 1), jnp.float32),
                  jax.ShapeDtypeStruct((STEPS, T, 1), jnp.int32),
                  jax.ShapeDtypeStruct((1, 16), jnp.int32))
    out_specs = (pl.BlockSpec((1, T, 256), lambda i: (i, 0, 0)),
                 pl.BlockSpec((1, G, 1), lambda i: (i, 0, 0)),
                 pl.BlockSpec((1, T, 1), lambda i: (i, 0, 0)),
                 pl.BlockSpec((1, T, 1), lambda i: (i, 0, 0)),
                 pl.BlockSpec((1, 16), lambda i: (0, 0)))

    return pl.pallas_call(
        _body_a,
        grid=(STEPS,),
        in_specs=specs,
        out_specs=out_specs,
        out_shape=out_shapes,
        scratch_shapes=[pltpu.VMEM((2, T // 2, T // 2), jnp.float32),
                        pltpu.VMEM((2, T // 2, T // 2), jnp.float32),
                        pltpu.VMEM((1, NS), jnp.float32)],
    )(*ins)


def _sc_dispatch(feats, species_flat, rank_i, offs16):
    mesh = plsc.VectorSubcoreMesh(core_axis_name="c", subcore_axis_name="s")
    NB, RB = 4, 64

    @functools.partial(
        pl.kernel, mesh=mesh,
        out_type=jax.ShapeDtypeStruct((NT, 256), jnp.int32),
        compiler_params=pltpu.CompilerParams(needs_layout_passes=False),
        scratch_types=[
            pltpu.VMEM((TB,), jnp.int32),
            pltpu.VMEM((TB,), jnp.int32),
            pltpu.VMEM((16,), jnp.int32),
            pltpu.VMEM((NB, RB), jnp.int32),
            pltpu.VMEM((3, RB, 256), jnp.int32),
            pltpu.SemaphoreType.DMA,
            pltpu.SemaphoreType.DMA,
        ])
    def k(ft_hbm, sp_hbm, rk_hbm, off_hbm, out_hbm,
          sp_v, rk_v, off_v, dest_v, rows_v, sem_i, sem_o):
        wid = lax.axis_index("s") * 2 + lax.axis_index("c")
        base = wid * TB
        ih, oh = {}, {}

        def start_in(b):
            ih[b] = pltpu.async_copy(
                ft_hbm.at[pl.ds(base + b * RB, RB)], rows_v.at[b % 3], sem_i)

        for b in range(3):
            start_in(b)
        pltpu.sync_copy(sp_hbm.at[pl.ds(base, TB)], sp_v)
        pltpu.sync_copy(rk_hbm.at[pl.ds(base, TB)], rk_v)
        pltpu.sync_copy(off_hbm, off_v)
        for j in range(16):
            s = sp_v[pl.ds(j * 16, 16)]
            r = rk_v[pl.ds(j * 16, 16)]
            o = plsc.load_gather(off_v, [s])
            dest_v[j // 4, pl.ds((j % 4) * 16, 16)] = o + r
        for b in range(NB):
            ih[b].wait()
            oh[b] = pltpu.async_copy(
                rows_v.at[b % 3], out_hbm.at[dest_v.at[b]], sem_o)
            if b + 3 < NB:
                oh[b].wait()
                start_in(b + 3)
        for b in range(NB):
            if not (b + 3 < NB):
                oh[b].wait()

    return k(feats, species_flat, rank_i, offs16)


def _body_b(sf_ref, cnt_ref, coul_ref,
            w0_ref, w1_ref, w2_ref, w3_ref,
            b0_ref, b1_ref, b2_ref, b3_ref,
            out_ref):
    b = pl.program_id(0)

    @pl.when(b == 0)
    def _init():
        out_ref[...] = coul_ref[...]

    pk = sf_ref[...]
    f_lo = jax.lax.bitcast_convert_type(
        pk & jnp.int32(-65536), jnp.float32)
    f_hi = jax.lax.bitcast_convert_type(
        jax.lax.shift_left(pk, 16), jnp.float32)
    molid = f_hi[:, 194:195]

    offs = cnt_ref[...].astype(jnp.float32)[:, 0:NS + 1]
    glob = (TB * b + jax.lax.broadcasted_iota(jnp.int32, (TB, 1), 0)
            ).astype(jnp.float32)
    lo_f = jnp.float32(TB) * b.astype(jnp.float32)
    hi_f = lo_f + jnp.float32(TB - 1)
    off8 = offs[:, 0:NS]
    e_lo = (jnp.sum(jnp.where(off8 <= lo_f, 1.0, 0.0)) - 1.0).astype(jnp.int32)
    e_hi = (jnp.sum(jnp.where(off8 <= hi_f, 1.0, 0.0)) - 1.0).astype(jnp.int32)

    def expert(i, acc):
        e = e_lo + i
        seg_lo = jnp.sum(jnp.where(
            jax.lax.broadcasted_iota(jnp.int32, (1, NS + 1), 1) == e,
            offs, 0.0))
        seg_hi = jnp.sum(jnp.where(
            jax.lax.broadcasted_iota(jnp.int32, (1, NS + 1), 1) == e + 1,
            offs, 0.0))
        msk = jnp.logical_and(glob >= seg_lo, glob < seg_hi)
        h = _celu(jnp.dot(f_lo, w0_ref[e, pl.ds(0, 256)])
                  + jnp.dot(f_hi, w0_ref[e, pl.ds(256, 256)]) + b0_ref[e])
        h = _celu(jnp.dot(h, w1_ref[e]) + b1_ref[e])
        h = _celu(jnp.dot(h, w2_ref[e]) + b2_ref[e])
        o = jnp.dot(h, w3_ref[e]) + b3_ref[e]
        return acc + jnp.where(msk, o, 0.0)

    en = jax.lax.fori_loop(0, e_hi - e_lo + 1, expert,
                           jnp.zeros((TB, 1), jnp.float32))
    mi = jax.lax.broadcasted_iota(jnp.int32, (TB, NM), 1).astype(jnp.float32)
    oh = jnp.where(molid == mi, 1.0, 0.0).astype(jnp.float32)
    out_ref[...] += jnp.sum(en * oh, axis=0, keepdims=True)


def _run_b(sorted_feats, offs, coul, params):
    p = params
    w0 = jnp.concatenate(
        [p['ani_W0'], jnp.zeros((NS, DF - 450, p['ani_W0'].shape[2]),
                                jnp.float32)], axis=1)

    def bs(a):
        nd = a.ndim
        return pl.BlockSpec(a.shape, lambda i, _n=nd: (0,) * _n)

    ins = [sorted_feats, offs, coul,
           w0, p['ani_W1'], p['ani_W2'], p['ani_W3'],
           p['ani_b0'][:, None, :], p['ani_b1'][:, None, :],
           p['ani_b2'][:, None, :], p['ani_b3'][:, None, :]]
    specs = [pl.BlockSpec((TB, 256), lambda i: (i, 0))]
    specs += [bs(a) for a in ins[1:]]

    out = pl.pallas_call(
        _body_b,
        grid=(NT // TB,),
        in_specs=specs,
        out_specs=pl.BlockSpec((1, NM), lambda i: (0, 0)),
        out_shape=jax.ShapeDtypeStruct((1, NM), jnp.float32),
    )(*ins)
    return out


def kernel(species, coordinates, net_charge, params):
    feats, coul, q2, rank_f, offs = _run_a(
        species, coordinates, net_charge, params)
    feats2d = feats.reshape(NT, 256)
    rank_i = rank_f.reshape(NT)
    offs16 = offs.reshape(16)
    sorted_feats = _sc_dispatch(feats2d, species.reshape(NT), rank_i, offs16)
    mol_e = _run_b(sorted_feats, offs, coul.reshape(1, NM), params)
    return species, mol_e.reshape(NM), q2.reshape(NM, NA)
